# initial kernel scaffold (unmeasured)
import jax
import jax.numpy as jnp
from jax import lax
from jax.experimental import pallas as pl
from jax.experimental.pallas import tpu as pltpu

N_DEV = 4


def kernel(x, w_mat, scale_x, scale_w):
    m_per, k = x.shape
    _, n_per = w_mat.shape
    m_tot = N_DEV * m_per

    x8 = x.astype(jnp.float8_e5m2)
    w8 = w_mat.astype(jnp.float8_e5m2)

    def body(scale_x_ref, scale_w_ref, x_ref, w_ref, out_ref,
             xg_ref, send_sems, recv_sems):
        my = lax.axis_index("i")
        s = scale_x_ref[0] * scale_w_ref[0]

        barrier = pltpu.get_barrier_semaphore()
        for o in range(1, N_DEV):
            pl.semaphore_signal(
                barrier, inc=1,
                device_id=((my + o) % N_DEV,),
                device_id_type=pl.DeviceIdType.MESH,
            )
        pl.semaphore_wait(barrier, N_DEV - 1)

        rdmas = []
        for o in range(1, N_DEV):
            rdma = pltpu.make_async_remote_copy(
                src_ref=x_ref,
                dst_ref=xg_ref.at[o - 1],
                send_sem=send_sems.at[o - 1],
                recv_sem=recv_sems.at[o - 1],
                device_id=((my + o) % N_DEV,),
                device_id_type=pl.DeviceIdType.MESH,
            )
            rdma.start()
            rdmas.append(rdma)

        def chunk_out(x_chunk):
            acc = jnp.dot(x_chunk, w_ref[...],
                          preferred_element_type=jnp.float32)
            return jnp.maximum(acc * s, 0.0)

        out_ref[pl.ds(my * m_per, m_per), :] = chunk_out(x_ref[...])

        for o in (1, 3, 2):
            rdmas[o - 1].wait_recv()
            origin = (my - o) % N_DEV
            out_ref[pl.ds(origin * m_per, m_per), :] = chunk_out(
                xg_ref[o - 1])

        for o in range(1, N_DEV):
            rdmas[o - 1].wait_send()

    return pl.pallas_call(
        body,
        out_shape=jax.ShapeDtypeStruct((m_tot, n_per), jnp.float32),
        in_specs=[
            pl.BlockSpec(memory_space=pltpu.SMEM),
            pl.BlockSpec(memory_space=pltpu.SMEM),
            pl.BlockSpec(memory_space=pltpu.VMEM),
            pl.BlockSpec(memory_space=pltpu.VMEM),
        ],
        out_specs=pl.BlockSpec(memory_space=pltpu.VMEM),
        scratch_shapes=[
            pltpu.VMEM((N_DEV - 1, m_per, k), jnp.float8_e5m2),
            pltpu.SemaphoreType.DMA((N_DEV - 1,)),
            pltpu.SemaphoreType.DMA((N_DEV - 1,)),
        ],
        compiler_params=pltpu.CompilerParams(collective_id=0),
    )(scale_x, scale_w, x8, w8)


# baseline (device time: 175004 ns/iter reference)
import jax
import jax.numpy as jnp
from jax import lax
from jax.experimental import pallas as pl
from jax.experimental.pallas import tpu as pltpu

N_DEV = 4


def kernel(x, w_mat, scale_x, scale_w):
    m_per, k = x.shape
    _, n_per = w_mat.shape
    m_tot = N_DEV * m_per

    x8 = x.astype(jnp.float8_e5m2)
    w8 = w_mat.astype(jnp.float8_e5m2)

    def body(scale_x_ref, scale_w_ref, x_ref, w_ref, out_ref,
             xg_ref, send_sems, recv_sems):
        my = lax.axis_index("i")
        s = scale_x_ref[0] * scale_w_ref[0]

        barrier = pltpu.get_barrier_semaphore()
        for o in range(1, N_DEV):
            pl.semaphore_signal(
                barrier, inc=1,
                device_id=((my + o) % N_DEV,),
                device_id_type=pl.DeviceIdType.MESH,
            )
        pl.semaphore_wait(barrier, N_DEV - 1)

        rdmas = []
        for o in range(1, N_DEV):
            rdma = pltpu.make_async_remote_copy(
                src_ref=x_ref,
                dst_ref=xg_ref.at[o - 1],
                send_sem=send_sems.at[o - 1],
                recv_sem=recv_sems.at[o - 1],
                device_id=((my + o) % N_DEV,),
                device_id_type=pl.DeviceIdType.MESH,
            )
            rdma.start()
            rdmas.append(rdma)

        def chunk_out(x_chunk):
            acc = jnp.dot(x_chunk, w_ref[...],
                          preferred_element_type=jnp.float32)
            return jnp.maximum(acc * s, 0.0)

        out_ref[pl.ds(my * m_per, m_per), :] = chunk_out(x_ref[...])

        for o in (1, 3, 2):
            rdmas[o - 1].wait_recv()
            origin = (my - o) % N_DEV
            out_ref[pl.ds(origin * m_per, m_per), :] = chunk_out(
                xg_ref[o - 1])

        for o in range(1, N_DEV):
            rdmas[o - 1].wait_send()

    return pl.pallas_call(
        body,
        out_shape=jax.ShapeDtypeStruct((m_tot, n_per), jnp.float32),
        in_specs=[
            pl.BlockSpec(memory_space=pltpu.SMEM),
            pl.BlockSpec(memory_space=pltpu.SMEM),
            pl.BlockSpec(memory_space=pltpu.VMEM),
            pl.BlockSpec(memory_space=pltpu.VMEM),
        ],
        out_specs=pl.BlockSpec(memory_space=pltpu.VMEM),
        scratch_shapes=[
            pltpu.VMEM((N_DEV - 1, m_per, k), jnp.float8_e5m2),
            pltpu.SemaphoreType.DMA((N_DEV - 1,)),
            pltpu.SemaphoreType.DMA((N_DEV - 1,)),
        ],
        compiler_params=pltpu.CompilerParams(
            collective_id=0,
            vmem_limit_bytes=100 * 1024 * 1024,
        ),
    )(scale_x, scale_w, x8, w8)


# device time: 142660 ns/iter; 1.2267x vs baseline; 1.2267x over previous
import jax
import jax.numpy as jnp
from jax import lax
from jax.experimental import pallas as pl
from jax.experimental.pallas import tpu as pltpu

N_DEV = 4


def kernel(x, w_mat, scale_x, scale_w):
    m_per, k = x.shape
    _, n_per = w_mat.shape
    m_tot = N_DEV * m_per
    mh = m_per // 2
    mq = m_per // 4

    x8 = x.astype(jnp.float8_e5m2)
    w8 = w_mat.astype(jnp.float8_e5m2)

    def body(scale_x_ref, scale_w_ref, x_ref, w8_ref, out_ref,
             xg_ref, send_sems, recv_sems):
        my = lax.axis_index("i")
        left = (my - 1) % N_DEV
        right = (my + 1) % N_DEV
        s = scale_x_ref[0] * scale_w_ref[0]

        FROM_L, FROM_R, DIAG = 0, 1, 2

        barrier = pltpu.get_barrier_semaphore()
        for nbr in (left, right):
            pl.semaphore_signal(
                barrier, inc=1,
                device_id=(nbr,), device_id_type=pl.DeviceIdType.MESH,
            )
        pl.semaphore_wait(barrier, 2)

        def rdma(i, src, dst, dev):
            return pltpu.make_async_remote_copy(
                src_ref=src, dst_ref=dst,
                send_sem=send_sems.at[i], recv_sem=recv_sems.at[i],
                device_id=(dev,), device_id_type=pl.DeviceIdType.MESH,
            )

        hop1 = [
            rdma(0, x_ref.at[pl.ds(0, mh)], xg_ref.at[FROM_L, pl.ds(0, mh)], right),
            rdma(1, x_ref.at[pl.ds(mh, mh)], xg_ref.at[FROM_L, pl.ds(mh, mh)], right),
            rdma(2, x_ref.at[pl.ds(0, mh)], xg_ref.at[FROM_R, pl.ds(0, mh)], left),
            rdma(3, x_ref.at[pl.ds(mh, mh)], xg_ref.at[FROM_R, pl.ds(mh, mh)], left),
        ]
        for r in hop1:
            r.start()

        fwd = [
            rdma(4, xg_ref.at[FROM_L, pl.ds(0, mq)], xg_ref.at[DIAG, pl.ds(0, mq)], right),
            rdma(5, xg_ref.at[FROM_L, pl.ds(mq, mq)], xg_ref.at[DIAG, pl.ds(mq, mq)], right),
            rdma(6, xg_ref.at[FROM_R, pl.ds(mh, mq)], xg_ref.at[DIAG, pl.ds(mh, mq)], left),
            rdma(7, xg_ref.at[FROM_R, pl.ds(mh + mq, mq)], xg_ref.at[DIAG, pl.ds(mh + mq, mq)], left),
        ]

        def gemm(x_chunk, out_row, rows):
            acc = jnp.dot(x_chunk, w8_ref[...],
                          preferred_element_type=jnp.float32)
            out_ref[pl.ds(out_row, rows), :] = jnp.maximum(acc * s, 0.0)

        gemm(x_ref[...], my * m_per, m_per)

        hop1[0].wait_recv()
        fwd[0].start()
        fwd[1].start()
        gemm(xg_ref[FROM_L, pl.ds(0, mh)], left * m_per, mh)
        hop1[2].wait_recv()
        gemm(xg_ref[FROM_R, pl.ds(0, mh)], right * m_per, mh)
        hop1[1].wait_recv()
        gemm(xg_ref[FROM_L, pl.ds(mh, mh)], left * m_per + mh, mh)
        hop1[3].wait_recv()
        fwd[2].start()
        fwd[3].start()
        gemm(xg_ref[FROM_R, pl.ds(mh, mh)], right * m_per + mh, mh)

        diag_row = ((my + 2) % N_DEV) * m_per
        fwd[0].wait_recv()
        gemm(xg_ref[DIAG, pl.ds(0, mq)], diag_row, mq)
        fwd[2].wait_recv()
        gemm(xg_ref[DIAG, pl.ds(mh, mq)], diag_row + mh, mq)
        fwd[1].wait_recv()
        gemm(xg_ref[DIAG, pl.ds(mq, mq)], diag_row + mq, mq)
        fwd[3].wait_recv()
        gemm(xg_ref[DIAG, pl.ds(mh + mq, mq)], diag_row + mh + mq, mq)

        for r in hop1 + fwd:
            r.wait_send()

    return pl.pallas_call(
        body,
        out_shape=jax.ShapeDtypeStruct((m_tot, n_per), jnp.float32),
        in_specs=[
            pl.BlockSpec(memory_space=pltpu.SMEM),
            pl.BlockSpec(memory_space=pltpu.SMEM),
            pl.BlockSpec(memory_space=pltpu.VMEM),
            pl.BlockSpec(memory_space=pltpu.VMEM),
        ],
        out_specs=pl.BlockSpec(memory_space=pltpu.VMEM),
        scratch_shapes=[
            pltpu.VMEM((3, m_per, k), jnp.float8_e5m2),
            pltpu.SemaphoreType.DMA((8,)),
            pltpu.SemaphoreType.DMA((8,)),
        ],
        compiler_params=pltpu.CompilerParams(
            collective_id=0,
            vmem_limit_bytes=100 * 1024 * 1024,
        ),
    )(scale_x, scale_w, x8, w8)


# device time: 116644 ns/iter; 1.5003x vs baseline; 1.2230x over previous
import jax
import jax.numpy as jnp
from jax import lax
from jax.experimental import pallas as pl
from jax.experimental.pallas import tpu as pltpu

N_DEV = 4


def kernel(x, w_mat, scale_x, scale_w):
    m_per, k = x.shape
    _, n_per = w_mat.shape
    m_tot = N_DEV * m_per
    mh = m_per // 2
    mq = m_per // 4
    kq = k // 4

    x8 = x.astype(jnp.float8_e5m2)

    def body(scale_x_ref, scale_w_ref, x_ref, w_hbm, out_hbm,
             xg_ref, w8_ref, wst_ref, ost_ref,
             send_sems, recv_sems, wdma_sems, odma_sems):
        my = lax.axis_index("i")
        left = (my - 1) % N_DEV
        right = (my + 1) % N_DEV
        s = scale_x_ref[0] * scale_w_ref[0]

        FROM_L, FROM_R, DIAG = 0, 1, 2

        barrier = pltpu.get_barrier_semaphore()
        for nbr in (left, right):
            pl.semaphore_signal(
                barrier, inc=1,
                device_id=(nbr,), device_id_type=pl.DeviceIdType.MESH,
            )
        pl.semaphore_wait(barrier, 2)

        def rdma(i, src, dst, dev):
            return pltpu.make_async_remote_copy(
                src_ref=src, dst_ref=dst,
                send_sem=send_sems.at[i], recv_sem=recv_sems.at[i],
                device_id=(dev,), device_id_type=pl.DeviceIdType.MESH,
            )

        hop1 = [
            rdma(0, x_ref.at[pl.ds(0, mh)], xg_ref.at[FROM_L, pl.ds(0, mh)], right),
            rdma(1, x_ref.at[pl.ds(mh, mh)], xg_ref.at[FROM_L, pl.ds(mh, mh)], right),
            rdma(2, x_ref.at[pl.ds(0, mh)], xg_ref.at[FROM_R, pl.ds(0, mh)], left),
            rdma(3, x_ref.at[pl.ds(mh, mh)], xg_ref.at[FROM_R, pl.ds(mh, mh)], left),
        ]
        for r in hop1:
            r.start()

        fwd = [
            rdma(4, xg_ref.at[FROM_L, pl.ds(0, mq)], xg_ref.at[DIAG, pl.ds(0, mq)], right),
            rdma(5, xg_ref.at[FROM_L, pl.ds(mq, mq)], xg_ref.at[DIAG, pl.ds(mq, mq)], right),
            rdma(6, xg_ref.at[FROM_R, pl.ds(mh, mq)], xg_ref.at[DIAG, pl.ds(mh, mq)], left),
            rdma(7, xg_ref.at[FROM_R, pl.ds(mh + mq, mq)], xg_ref.at[DIAG, pl.ds(mh + mq, mq)], left),
        ]

        wcp = [
            pltpu.make_async_copy(
                w_hbm.at[pl.ds(q * kq, kq)], wst_ref.at[q % 2],
                wdma_sems.at[q % 2])
            for q in range(4)
        ]
        wcp[0].start()
        wcp[1].start()
        for q in range(4):
            wcp[q].wait()
            w8_ref[pl.ds(q * kq, kq), :] = wst_ref[q % 2].astype(
                jnp.float8_e5m2)
            if q + 2 < 4:
                wcp[q + 2].start()

        pending = [None, None]
        next_slot = [0]

        def gemm(x_chunk, out_row, rows):
            slot = next_slot[0]
            next_slot[0] ^= 1
            if pending[slot] is not None:
                pending[slot].wait()
            acc = jnp.dot(x_chunk, w8_ref[...],
                          preferred_element_type=jnp.float32)
            ost_ref[slot, pl.ds(0, rows)] = jnp.maximum(acc * s, 0.0)
            cp = pltpu.make_async_copy(
                ost_ref.at[slot, pl.ds(0, rows)],
                out_hbm.at[pl.ds(out_row, rows)],
                odma_sems.at[slot],
            )
            cp.start()
            pending[slot] = cp

        gemm(x_ref[pl.ds(0, mh)], my * m_per, mh)
        gemm(x_ref[pl.ds(mh, mh)], my * m_per + mh, mh)

        hop1[0].wait_recv()
        fwd[0].start()
        fwd[1].start()
        gemm(xg_ref[FROM_L, pl.ds(0, mh)], left * m_per, mh)
        hop1[2].wait_recv()
        gemm(xg_ref[FROM_R, pl.ds(0, mh)], right * m_per, mh)
        hop1[1].wait_recv()
        gemm(xg_ref[FROM_L, pl.ds(mh, mh)], left * m_per + mh, mh)
        hop1[3].wait_recv()
        fwd[2].start()
        fwd[3].start()
        gemm(xg_ref[FROM_R, pl.ds(mh, mh)], right * m_per + mh, mh)

        diag_row = ((my + 2) % N_DEV) * m_per
        fwd[0].wait_recv()
        gemm(xg_ref[DIAG, pl.ds(0, mq)], diag_row, mq)
        fwd[2].wait_recv()
        gemm(xg_ref[DIAG, pl.ds(mh, mq)], diag_row + mh, mq)
        fwd[1].wait_recv()
        gemm(xg_ref[DIAG, pl.ds(mq, mq)], diag_row + mq, mq)
        fwd[3].wait_recv()
        gemm(xg_ref[DIAG, pl.ds(mh + mq, mq)], diag_row + mh + mq, mq)

        for p in pending:
            p.wait()
        for r in hop1 + fwd:
            r.wait_send()

    return pl.pallas_call(
        body,
        out_shape=jax.ShapeDtypeStruct((m_tot, n_per), jnp.float32),
        in_specs=[
            pl.BlockSpec(memory_space=pltpu.SMEM),
            pl.BlockSpec(memory_space=pltpu.SMEM),
            pl.BlockSpec(memory_space=pltpu.VMEM),
            pl.BlockSpec(memory_space=pl.ANY),
        ],
        out_specs=pl.BlockSpec(memory_space=pl.ANY),
        scratch_shapes=[
            pltpu.VMEM((3, m_per, k), jnp.float8_e5m2),
            pltpu.VMEM((k, n_per), jnp.float8_e5m2),
            pltpu.VMEM((2, kq, n_per), jnp.float32),
            pltpu.VMEM((2, mh, n_per), jnp.float32),
            pltpu.SemaphoreType.DMA((8,)),
            pltpu.SemaphoreType.DMA((8,)),
            pltpu.SemaphoreType.DMA((2,)),
            pltpu.SemaphoreType.DMA((2,)),
        ],
        compiler_params=pltpu.CompilerParams(
            collective_id=0,
            vmem_limit_bytes=100 * 1024 * 1024,
        ),
    )(scale_x, scale_w, x8, w_mat)
